# baseline (speedup 1.0000x reference)
import jax
import jax.numpy as jnp
from jax.experimental import pallas as pl
from jax.experimental.pallas import tpu as pltpu

_SIZE = 65536
_SHIFT = 8192
_ZLEN = _SIZE - _SHIFT
_ROWS = 64


def _body(x_hbm, o_hbm, zbuf, xbuf, zsem, isem, osem):
    icp = pltpu.make_async_copy(x_hbm, xbuf, isem)
    icp.start()
    zbuf[...] = jnp.zeros_like(zbuf)
    zcps = [
        pltpu.make_async_copy(
            zbuf, o_hbm.at[pl.ds(r * _SIZE, _ZLEN)], zsem)
        for r in range(_ROWS)
    ]
    for c in zcps:
        c.start()
    icp.wait()
    wcps = [
        pltpu.make_async_copy(
            xbuf.at[pl.ds(r * _SHIFT, _SHIFT)],
            o_hbm.at[pl.ds(r * _SIZE + _ZLEN, _SHIFT)], osem)
        for r in range(_ROWS)
    ]
    for c in wcps:
        c.start()
    for c in zcps:
        c.wait()
    for c in wcps:
        c.wait()


def kernel(x):
    xf = x.reshape(_ROWS * _SHIFT)
    out = pl.pallas_call(
        _body,
        in_specs=[pl.BlockSpec(memory_space=pl.ANY)],
        out_specs=pl.BlockSpec(memory_space=pl.ANY),
        out_shape=jax.ShapeDtypeStruct((_ROWS * _SIZE,), jnp.float32),
        scratch_shapes=[
            pltpu.VMEM((_ZLEN,), jnp.float32),
            pltpu.VMEM((_ROWS * _SHIFT,), jnp.float32),
            pltpu.SemaphoreType.DMA,
            pltpu.SemaphoreType.DMA,
            pltpu.SemaphoreType.DMA,
        ],
    )(xf)
    return out.reshape(x.shape[:-1] + (_SIZE,))
